# baseline (speedup 1.0000x reference)
import functools

import jax
import jax.numpy as jnp
import numpy as np
from jax import lax
from jax.experimental import pallas as pl
from jax.experimental.pallas import tpu as pltpu
from jax.experimental.pallas import tpu_sc as plsc

MAX_SEQ = 2048
D_MODEL = 2048
VOCAB = 77
BATCH = 4
LANES = 16


@functools.lru_cache(maxsize=1)
def _pe_np():
    position = np.arange(MAX_SEQ, dtype=np.float32)[:, None]
    div_term = np.exp(
        np.arange(0, D_MODEL, 2, dtype=np.float32) * -(np.log(10000.0) / D_MODEL)
    )
    pe = np.zeros((MAX_SEQ, D_MODEL), dtype=np.float32)
    pe[:, 0::2] = np.sin(position * div_term)
    pe[:, 1::2] = np.cos(position * div_term)
    return pe


@functools.lru_cache(maxsize=1)
def _make_sc_kernel():
    info = plsc.get_sparse_core_info()
    NC, NS = info.num_cores, info.num_subcores
    NW = NC * NS
    N = BATCH * MAX_SEQ
    NB = 2
    NG = NW // NB
    SPW = MAX_SEQ // NG
    K = 8
    NSC = SPW // K
    NCH = NSC * NB
    mesh = plsc.VectorSubcoreMesh(core_axis_name="c", subcore_axis_name="s")

    @functools.partial(
        pl.kernel,
        mesh=mesh,
        out_type=jax.ShapeDtypeStruct((N, D_MODEL), jnp.float32),
        scratch_types=[
            pltpu.VMEM((SPW * NB,), jnp.int32),
            pltpu.VMEM((K, D_MODEL), jnp.float32),
            pltpu.VMEM((K, D_MODEL), jnp.float32),
            pltpu.VMEM((K, D_MODEL), jnp.float32),
            pltpu.VMEM((K, D_MODEL), jnp.float32),
            pltpu.VMEM((K, D_MODEL), jnp.float32),
            pltpu.SemaphoreType.DMA,
            pltpu.SemaphoreType.DMA,
            pltpu.SemaphoreType.DMA,
            pltpu.SemaphoreType.DMA,
            pltpu.SemaphoreType.DMA,
            pltpu.SemaphoreType.DMA,
            pltpu.SemaphoreType.DMA,
            pltpu.SemaphoreType.DMA,
        ],
    )
    def k(x_hbm, table_hbm, pe_hbm, out_hbm, idx_v,
          e0, e1, e2, q0, q1, g0, g1, g2, p0, p1, o0, o1, o2):
        emb = (e0, e1, e2)
        peb = (q0, q1)
        gs = (g0, g1, g2)
        ps = (p0, p1)
        osem = (o0, o1, o2)
        wid = lax.axis_index("s") * NC + lax.axis_index("c")
        pltpu.sync_copy(x_hbm.at[wid], idx_v)
        s0 = (wid % NG) * SPW
        b0 = (wid // NG) * NB

        def start_g(c):
            b = c % 3
            return pltpu.async_copy(
                table_hbm.at[idx_v.at[pl.ds(c * K, K)]], emb[b], gs[b]
            )

        def start_pe(sc):
            return pltpu.async_copy(
                pe_hbm.at[pl.ds(s0 + sc * K, K)], peb[sc % 2], ps[sc % 2]
            )

        g_flight = {0: start_g(0), 1: start_g(1)}
        pe_flight = {0: start_pe(0)}
        out_flight = {}
        for sc in range(NSC):
            qb = sc % 2
            if sc + 1 < NSC:
                pe_flight[sc + 1] = start_pe(sc + 1)
            pe_flight.pop(sc).wait()
            for bt in range(NB):
                c = sc * NB + bt
                b = c % 3
                if c + 2 < NCH:
                    if c - 1 in out_flight:
                        out_flight.pop(c - 1).wait()
                    g_flight[c + 2] = start_g(c + 2)
                g_flight.pop(c).wait()

                def body(i, _):
                    col = i * LANES
                    for r in range(K):
                        v = peb[qb][r, pl.ds(col, LANES)]
                        plsc.addupdate(emb[b].at[r, pl.ds(col, LANES)], v)
                    return 0

                lax.fori_loop(0, D_MODEL // LANES, body, 0)
                out_flight[c] = pltpu.async_copy(
                    emb[b],
                    out_hbm.at[pl.ds((b0 + bt) * MAX_SEQ + s0 + sc * K, K)],
                    osem[b],
                )
        for c in sorted(out_flight):
            out_flight.pop(c).wait()

    return k


def kernel(x, embedding, start_token, end_token):
    del start_token, end_token
    info = plsc.get_sparse_core_info()
    NW = info.num_cores * info.num_subcores
    NB = 2
    NG = NW // NB
    SPW = MAX_SEQ // NG
    K = 8
    x_r = (
        x.astype(jnp.int32)
        .reshape(NW // NG, NB, NG, SPW // K, K)
        .transpose(0, 2, 3, 1, 4)
        .reshape(NW, SPW * NB)
    )
    pe = jnp.asarray(_pe_np())
    out = _make_sc_kernel()(x_r, embedding, pe)
    return out.reshape(BATCH, MAX_SEQ, D_MODEL)


# speedup vs baseline: 1.4089x; 1.4089x over previous
import functools

import jax
import jax.numpy as jnp
import numpy as np
from jax import lax
from jax.experimental import pallas as pl
from jax.experimental.pallas import tpu as pltpu
from jax.experimental.pallas import tpu_sc as plsc

MAX_SEQ = 2048
D_MODEL = 2048
VOCAB = 77
BATCH = 4
LANES = 16


@functools.lru_cache(maxsize=1)
def _pe_np():
    position = np.arange(MAX_SEQ, dtype=np.float32)[:, None]
    div_term = np.exp(
        np.arange(0, D_MODEL, 2, dtype=np.float32) * -(np.log(10000.0) / D_MODEL)
    )
    pe = np.zeros((MAX_SEQ, D_MODEL), dtype=np.float32)
    pe[:, 0::2] = np.sin(position * div_term)
    pe[:, 1::2] = np.cos(position * div_term)
    return pe


@functools.lru_cache(maxsize=1)
def _make_sc_kernel():
    info = plsc.get_sparse_core_info()
    NC, NS = info.num_cores, info.num_subcores
    NW = NC * NS
    N = BATCH * MAX_SEQ
    NB = 2
    NG = NW // NB
    SPW = MAX_SEQ // NG
    K = 8
    NSC = SPW // K
    NCH = NSC * NB
    mesh = plsc.VectorSubcoreMesh(core_axis_name="c", subcore_axis_name="s")

    @functools.partial(
        pl.kernel,
        mesh=mesh,
        out_type=jax.ShapeDtypeStruct((N, D_MODEL), jnp.float32),
        scratch_types=[
            pltpu.VMEM((NCH, K), jnp.int32),
            pltpu.VMEM((K, D_MODEL), jnp.float32),
            pltpu.VMEM((K, D_MODEL), jnp.float32),
            pltpu.VMEM((K, D_MODEL), jnp.float32),
            pltpu.VMEM((K, D_MODEL), jnp.float32),
            pltpu.VMEM((K, D_MODEL), jnp.float32),
            pltpu.SemaphoreType.DMA,
            pltpu.SemaphoreType.DMA,
            pltpu.SemaphoreType.DMA,
            pltpu.SemaphoreType.DMA,
            pltpu.SemaphoreType.DMA,
            pltpu.SemaphoreType.DMA,
            pltpu.SemaphoreType.DMA,
            pltpu.SemaphoreType.DMA,
        ],
    )
    def k(x_hbm, table_hbm, pe_hbm, out_hbm, idx_v,
          e0, e1, e2, q0, q1, g0, g1, g2, p0, p1, o0, o1, o2):
        emb = (e0, e1, e2)
        peb = (q0, q1)
        gs = (g0, g1, g2)
        ps = (p0, p1)
        osem = (o0, o1, o2)
        wid = lax.axis_index("s") * NC + lax.axis_index("c")
        pltpu.sync_copy(x_hbm.at[wid], idx_v)
        s0 = (wid % NG) * SPW
        b0 = (wid // NG) * NB

        def start_g(c):
            b = c % 3
            return pltpu.async_copy(
                table_hbm.at[idx_v.at[c]], emb[b], gs[b]
            )

        def start_pe(sc):
            return pltpu.async_copy(
                pe_hbm.at[pl.ds(s0 + sc * K, K)], peb[sc % 2], ps[sc % 2]
            )

        g_flight = {0: start_g(0), 1: start_g(1)}
        pe_flight = {0: start_pe(0)}
        out_flight = {}
        for sc in range(NSC):
            qb = sc % 2
            if sc + 1 < NSC:
                pe_flight[sc + 1] = start_pe(sc + 1)
            pe_flight.pop(sc).wait()
            for bt in range(NB):
                c = sc * NB + bt
                b = c % 3
                if c + 2 < NCH:
                    if c - 1 in out_flight:
                        out_flight.pop(c - 1).wait()
                    g_flight[c + 2] = start_g(c + 2)
                g_flight.pop(c).wait()

                def body(i, _):
                    col = i * LANES
                    for r in range(K):
                        v = peb[qb][r, pl.ds(col, LANES)]
                        plsc.addupdate(emb[b].at[r, pl.ds(col, LANES)], v)
                    return 0

                lax.fori_loop(0, D_MODEL // LANES, body, 0)
                out_flight[c] = pltpu.async_copy(
                    emb[b],
                    out_hbm.at[pl.ds((b0 + bt) * MAX_SEQ + s0 + sc * K, K)],
                    osem[b],
                )
        for c in sorted(out_flight):
            out_flight.pop(c).wait()

    return k


def kernel(x, embedding, start_token, end_token):
    del start_token, end_token
    info = plsc.get_sparse_core_info()
    NW = info.num_cores * info.num_subcores
    NB = 2
    NG = NW // NB
    SPW = MAX_SEQ // NG
    K = 8
    x_r = (
        x.astype(jnp.int32)
        .reshape(NW // NG, NB, NG, SPW // K, K)
        .transpose(0, 2, 3, 1, 4)
        .reshape(NW, SPW * NB // K, K)
    )
    pe = jnp.asarray(_pe_np())
    out = _make_sc_kernel()(x_r, embedding, pe)
    return out.reshape(BATCH, MAX_SEQ, D_MODEL)


# speedup vs baseline: 1.4467x; 1.0268x over previous
import functools

import jax
import jax.numpy as jnp
import numpy as np
from jax import lax
from jax.experimental import pallas as pl
from jax.experimental.pallas import tpu as pltpu
from jax.experimental.pallas import tpu_sc as plsc

MAX_SEQ = 2048
D_MODEL = 2048
VOCAB = 77
BATCH = 4
LANES = 16


@functools.lru_cache(maxsize=1)
def _pe_np():
    position = np.arange(MAX_SEQ, dtype=np.float32)[:, None]
    div_term = np.exp(
        np.arange(0, D_MODEL, 2, dtype=np.float32) * -(np.log(10000.0) / D_MODEL)
    )
    pe = np.zeros((MAX_SEQ, D_MODEL), dtype=np.float32)
    pe[:, 0::2] = np.sin(position * div_term)
    pe[:, 1::2] = np.cos(position * div_term)
    return pe


@functools.lru_cache(maxsize=1)
def _make_sc_kernel():
    info = plsc.get_sparse_core_info()
    NC, NS = info.num_cores, info.num_subcores
    NW = NC * NS
    N = BATCH * MAX_SEQ
    SPW = MAX_SEQ // NW
    K = 8
    NSC = SPW // K
    NCH = NSC * BATCH
    mesh = plsc.VectorSubcoreMesh(core_axis_name="c", subcore_axis_name="s")

    @functools.partial(
        pl.kernel,
        mesh=mesh,
        out_type=jax.ShapeDtypeStruct((N, D_MODEL), jnp.float32),
        scratch_types=[
            pltpu.VMEM((NCH, K), jnp.int32),
            pltpu.VMEM((K, D_MODEL), jnp.float32),
            pltpu.VMEM((K, D_MODEL), jnp.float32),
            pltpu.VMEM((K, D_MODEL), jnp.float32),
            pltpu.VMEM((K, D_MODEL), jnp.float32),
            pltpu.VMEM((K, D_MODEL), jnp.float32),
            pltpu.SemaphoreType.DMA,
            pltpu.SemaphoreType.DMA,
            pltpu.SemaphoreType.DMA,
            pltpu.SemaphoreType.DMA,
            pltpu.SemaphoreType.DMA,
            pltpu.SemaphoreType.DMA,
            pltpu.SemaphoreType.DMA,
            pltpu.SemaphoreType.DMA,
        ],
    )
    def k(x_hbm, table_hbm, pe_hbm, out_hbm, idx_v,
          e0, e1, e2, q0, q1, g0, g1, g2, p0, p1, o0, o1, o2):
        emb = (e0, e1, e2)
        peb = (q0, q1)
        gs = (g0, g1, g2)
        ps = (p0, p1)
        osem = (o0, o1, o2)
        wid = lax.axis_index("s") * NC + lax.axis_index("c")
        pltpu.sync_copy(x_hbm.at[wid], idx_v)
        s0 = wid * SPW

        def start_g(c):
            b = c % 3
            return pltpu.async_copy(
                table_hbm.at[idx_v.at[c]], emb[b], gs[b]
            )

        def start_pe(sc):
            return pltpu.async_copy(
                pe_hbm.at[pl.ds(s0 + sc * K, K)], peb[sc % 2], ps[sc % 2]
            )

        g_flight = {0: start_g(0), 1: start_g(1)}
        pe_flight = {0: start_pe(0)}
        out_flight = {}
        for sc in range(NSC):
            qb = sc % 2
            if sc + 1 < NSC:
                pe_flight[sc + 1] = start_pe(sc + 1)
            pe_flight.pop(sc).wait()
            for bt in range(BATCH):
                c = sc * BATCH + bt
                b = c % 3
                if c + 2 < NCH:
                    if c - 1 in out_flight:
                        out_flight.pop(c - 1).wait()
                    g_flight[c + 2] = start_g(c + 2)
                g_flight.pop(c).wait()

                def body(i, _):
                    col = i * LANES
                    for r in range(K):
                        v = peb[qb][r, pl.ds(col, LANES)]
                        plsc.addupdate(emb[b].at[r, pl.ds(col, LANES)], v)
                    return 0

                lax.fori_loop(0, D_MODEL // LANES, body, 0)
                out_flight[c] = pltpu.async_copy(
                    emb[b],
                    out_hbm.at[pl.ds(bt * MAX_SEQ + s0 + sc * K, K)],
                    osem[b],
                )
        for c in sorted(out_flight):
            out_flight.pop(c).wait()

    return k


def kernel(x, embedding, start_token, end_token):
    del start_token, end_token
    info = plsc.get_sparse_core_info()
    NW = info.num_cores * info.num_subcores
    SPW = MAX_SEQ // NW
    K = 8
    x_r = (
        x.astype(jnp.int32)
        .reshape(BATCH, NW, SPW // K, K)
        .transpose(1, 2, 0, 3)
        .reshape(NW, SPW * BATCH // K, K)
    )
    pe = jnp.asarray(_pe_np())
    out = _make_sc_kernel()(x_r, embedding, pe)
    return out.reshape(BATCH, MAX_SEQ, D_MODEL)
